# baseline (speedup 1.0000x reference)
import functools

import jax
import jax.numpy as jnp
from jax import lax
from jax.experimental import pallas as pl
from jax.experimental.pallas import tpu as pltpu
from jax.experimental.pallas import tpu_sc as plsc

FEAT = 64
SIZE = 512
LANES = 16
CHUNK = 64
NBUF = 2


def _pixel(c):
    p = (c + 1.0) * 0.5 * (SIZE - 1)
    i0 = p.astype(jnp.int32)
    f = p - i0.astype(jnp.float32)
    i0 = jnp.minimum(jnp.maximum(i0, 0), SIZE - 1)
    i1 = jnp.minimum(i0 + 1, SIZE - 1)
    return i0, i1, f


def _tc_transpose_tables(p_xy, p_xz, p_yz):
    cols = 4096
    grid = (SIZE * SIZE // cols,)

    def body(xy_ref, xz_ref, yz_ref, oxy_ref, oxz_ref, oyz_ref):
        oxy_ref[...] = xy_ref[...].T
        oxz_ref[...] = xz_ref[...].T
        oyz_ref[...] = yz_ref[...].T

    in_spec = pl.BlockSpec((FEAT, cols), lambda i: (0, i))
    out_spec = pl.BlockSpec((cols, FEAT), lambda i: (i, 0))
    out_sds = jax.ShapeDtypeStruct((SIZE * SIZE, FEAT), jnp.float32)
    return pl.pallas_call(
        body,
        grid=grid,
        in_specs=[in_spec] * 3,
        out_specs=[out_spec] * 3,
        out_shape=[out_sds] * 3,
    )(p_xy, p_xz, p_yz)


def _sc_embed(xt, txy, txz, tyz, B, nw):
    pts_per_w = B // nw
    nch = pts_per_w // CHUNK
    mesh = plsc.VectorSubcoreMesh(core_axis_name="c", subcore_axis_name="s")

    @functools.partial(
        pl.kernel,
        out_type=jax.ShapeDtypeStruct((B, FEAT), jnp.float32),
        mesh=mesh,
        compiler_params=pltpu.CompilerParams(
            needs_layout_passes=False, use_tc_tiling_on_sc=False),
        scratch_types=[
            pltpu.VMEM((NBUF, 3 * CHUNK), jnp.float32),
            pltpu.VMEM((NBUF, 12, CHUNK), jnp.int32),
            pltpu.VMEM((NBUF, LANES, CHUNK), jnp.float32),
            pltpu.VMEM((NBUF, 12, CHUNK, FEAT), jnp.float32),
            pltpu.VMEM((NBUF, CHUNK, FEAT), jnp.float32),
            pltpu.SemaphoreType.DMA,
            pltpu.SemaphoreType.DMA,
            pltpu.SemaphoreType.DMA,
        ],
    )
    def k(xt_h, txy_h, txz_h, tyz_h, out_h, coords, idx, wgt, rows, obuf,
          sem0, sem1, semc):
        wid = lax.axis_index("s") * 2 + lax.axis_index("c")
        w_base = wid * pts_per_w
        tabs = (txy_h, txz_h, tyz_h)
        sems = (sem0, sem1)

        iota3 = lax.iota(jnp.int32, LANES) * 3

        def compute_idx(kk, b):
            base = w_base + kk * CHUNK
            cp = pltpu.make_async_copy(xt_h.at[pl.ds(base * 3, CHUNK * 3)],
                                       coords.at[b], semc)
            cp.start()
            cp.wait()
            for j in range(CHUNK // LANES):
                sl = pl.ds(j * LANES, LANES)
                iw = [None] * 3
                ih = [None] * 3
                fr = [None] * 3
                for d in range(3):
                    cd = plsc.load_gather(coords.at[b],
                                          [iota3 + (j * LANES * 3 + d)])
                    iw[d], ih[d], fr[d] = _pixel(cd)
                for p, (dw, dh) in enumerate(((0, 1), (0, 2), (1, 2))):
                    w0, w1, fw = iw[dw], ih[dw], fr[dw]
                    h0, h1, fh = iw[dh], ih[dh], fr[dh]
                    r00 = h0 * SIZE + w0
                    r01 = h0 * SIZE + w1
                    r10 = h1 * SIZE + w0
                    r11 = h1 * SIZE + w1
                    gw = 1.0 - fw
                    gh = 1.0 - fh
                    idx[b, 4 * p + 0, sl] = r00
                    idx[b, 4 * p + 1, sl] = r01
                    idx[b, 4 * p + 2, sl] = r10
                    idx[b, 4 * p + 3, sl] = r11
                    wgt[b, 4 * p + 0, sl] = gh * gw
                    wgt[b, 4 * p + 1, sl] = gh * fw
                    wgt[b, 4 * p + 2, sl] = fh * gw
                    wgt[b, 4 * p + 3, sl] = fh * fw

        def start_gathers(b):
            for p in range(3):
                for c in range(4):
                    s = 4 * p + c
                    pltpu.async_copy(tabs[p].at[idx.at[b, s]],
                                     rows.at[b, s], sems[b])

        def wait_gathers(b):
            for p in range(3):
                for c in range(4):
                    s = 4 * p + c
                    pltpu.make_async_copy(tabs[p].at[idx.at[b, s]],
                                          rows.at[b, s], sems[b]).wait()

        def combine(b):
            iota16 = lax.iota(jnp.int32, LANES)

            def body(g, _):
                wv = plsc.load_gather(
                    wgt.at[b], [iota16, jnp.full((LANES,), 0, jnp.int32) + g])
                for q in range(FEAT // LANES):
                    sl = pl.ds(q * LANES, LANES)
                    acc = wv[0] * rows[b, 0, g, sl]
                    for s in range(1, 12):
                        acc = acc + wv[s] * rows[b, s, g, sl]
                    obuf[b, g, sl] = acc
                return 0
            lax.fori_loop(0, CHUNK, body, 0)

        compute_idx(0, 0)
        start_gathers(0)

        def outer(i, _):
            for b in range(NBUF):
                kk = i * NBUF + b
                nb = (b + 1) % NBUF

                @pl.when(kk + 1 < nch)
                def _():
                    compute_idx(kk + 1, nb)
                    start_gathers(nb)

                wait_gathers(b)
                combine(b)
                pltpu.sync_copy(obuf.at[b],
                                out_h.at[pl.ds(w_base + kk * CHUNK, CHUNK)])
            return 0

        lax.fori_loop(0, nch // NBUF, outer, 0)

    return k(xt, txy, txz, tyz)


def kernel(x, rays_o, xy_plane, yz_plane, xz_plane):
    del rays_o
    B = x.shape[0]
    info = plsc.get_sparse_core_info()
    nw = info.num_cores * info.num_subcores
    xflat = x.reshape(B * 3)

    def flat(plane):
        return plane.reshape(FEAT, SIZE * SIZE)

    txy, txz, tyz = _tc_transpose_tables(flat(xy_plane), flat(xz_plane),
                                         flat(yz_plane))
    return _sc_embed(xflat, txy, txz, tyz, B, nw)


# speedup vs baseline: 1.6805x; 1.6805x over previous
import functools

import jax
import jax.numpy as jnp
from jax import lax
from jax.experimental import pallas as pl
from jax.experimental.pallas import tpu as pltpu
from jax.experimental.pallas import tpu_sc as plsc

FEAT = 64
SIZE = 512
LANES = 16
CHUNK = 64
NBUF = 2


def _pixel(c):
    p = (c + 1.0) * 0.5 * (SIZE - 1)
    i0 = p.astype(jnp.int32)
    f = p - i0.astype(jnp.float32)
    i0 = jnp.minimum(jnp.maximum(i0, 0), SIZE - 1)
    i1 = jnp.minimum(i0 + 1, SIZE - 1)
    return i0, i1, f


def _sc_embed(xt, txy, txz, tyz, B, nw):
    pts_per_w = B // nw
    nch = pts_per_w // CHUNK
    mesh = plsc.VectorSubcoreMesh(core_axis_name="c", subcore_axis_name="s")

    @functools.partial(
        pl.kernel,
        out_type=jax.ShapeDtypeStruct((B, FEAT), jnp.float32),
        mesh=mesh,
        compiler_params=pltpu.CompilerParams(
            needs_layout_passes=False, use_tc_tiling_on_sc=False),
        scratch_types=[
            pltpu.VMEM((NBUF, 3, CHUNK), jnp.float32),
            pltpu.VMEM((NBUF, 12, CHUNK), jnp.int32),
            pltpu.VMEM((NBUF, LANES, CHUNK + 1), jnp.float32),
            pltpu.VMEM((NBUF, 12, CHUNK, FEAT), jnp.float32),
            pltpu.VMEM((NBUF, CHUNK, FEAT), jnp.float32),
            pltpu.SemaphoreType.DMA,
            pltpu.SemaphoreType.DMA,
            pltpu.SemaphoreType.DMA,
            pltpu.SemaphoreType.DMA,
            pltpu.SemaphoreType.DMA,
        ],
    )
    def k(xt_h, txy_h, txz_h, tyz_h, out_h, coords, idx, wgt, rows, obuf,
          sem0, sem1, semc, semo0, semo1):
        wid = lax.axis_index("s") * 2 + lax.axis_index("c")
        w_base = wid * pts_per_w
        tabs = (txy_h, txz_h, tyz_h)
        sems = (sem0, sem1)
        semos = (semo0, semo1)
        iota16 = lax.iota(jnp.int32, LANES)

        def out_copy(kk, b):
            return pltpu.make_async_copy(
                obuf.at[b], out_h.at[pl.ds(w_base + kk * CHUNK, CHUNK)],
                semos[b])

        def compute_idx(kk, b):
            base = w_base + kk * CHUNK
            for d in range(3):
                pltpu.async_copy(xt_h.at[d, pl.ds(base, CHUNK)],
                                 coords.at[b, d], semc)
            for d in range(3):
                pltpu.make_async_copy(xt_h.at[d, pl.ds(base, CHUNK)],
                                      coords.at[b, d], semc).wait()
            for j in range(CHUNK // LANES):
                sl = pl.ds(j * LANES, LANES)
                iw = [None] * 3
                ih = [None] * 3
                fr = [None] * 3
                for d in range(3):
                    iw[d], ih[d], fr[d] = _pixel(coords[b, d, sl])
                for p, (dw, dh) in enumerate(((0, 1), (0, 2), (1, 2))):
                    w0, w1, fw = iw[dw], ih[dw], fr[dw]
                    h0, h1, fh = iw[dh], ih[dh], fr[dh]
                    gw = 1.0 - fw
                    gh = 1.0 - fh
                    idx[b, 4 * p + 0, sl] = h0 * SIZE + w0
                    idx[b, 4 * p + 1, sl] = h0 * SIZE + w1
                    idx[b, 4 * p + 2, sl] = h1 * SIZE + w0
                    idx[b, 4 * p + 3, sl] = h1 * SIZE + w1
                    wgt[b, 4 * p + 0, sl] = gh * gw
                    wgt[b, 4 * p + 1, sl] = gh * fw
                    wgt[b, 4 * p + 2, sl] = fh * gw
                    wgt[b, 4 * p + 3, sl] = fh * fw

        def start_gathers(b):
            for s in range(12):
                pltpu.async_copy(tabs[s // 4].at[idx.at[b, s]],
                                 rows.at[b, s], sems[b])

        def wait_gathers(b):
            for s in range(12):
                pltpu.make_async_copy(tabs[s // 4].at[idx.at[b, s]],
                                      rows.at[b, s], sems[b]).wait()

        def combine(b):
            def body(g, _):
                gs = jnp.full((LANES,), 0, jnp.int32) + g
                wv = plsc.load_gather(wgt.at[b], [iota16, gs])
                for q in range(FEAT // LANES):
                    sl = pl.ds(q * LANES, LANES)
                    terms = [wv[s] * rows[b, s, g, sl] for s in range(12)]
                    while len(terms) > 1:
                        terms = [terms[i] + terms[i + 1]
                                 for i in range(0, len(terms) - 1, 2)] + (
                                     [terms[-1]] if len(terms) % 2 else [])
                    obuf[b, g, sl] = terms[0]
                return 0
            lax.fori_loop(0, CHUNK, body, 0)

        compute_idx(0, 0)
        start_gathers(0)

        def outer(i, _):
            for b in range(NBUF):
                kk = i * NBUF + b
                nb = (b + 1) % NBUF

                @pl.when(kk + 1 < nch)
                def _():
                    compute_idx(kk + 1, nb)
                    start_gathers(nb)

                wait_gathers(b)

                @pl.when(kk >= NBUF)
                def _():
                    out_copy(kk - NBUF, b).wait()

                combine(b)
                out_copy(kk, b).start()
            return 0

        lax.fori_loop(0, nch // NBUF, outer, 0)
        for b in range(NBUF):
            out_copy(nch - NBUF + b, b).wait()

    return k(xt, txy, txz, tyz)


def kernel(x, rays_o, xy_plane, yz_plane, xz_plane):
    del rays_o
    B = x.shape[0]
    info = plsc.get_sparse_core_info()
    nw = info.num_cores * info.num_subcores
    xt = x.T

    def tab(plane):
        return plane[0].transpose(1, 2, 0).reshape(SIZE * SIZE, FEAT)

    return _sc_embed(xt, tab(xy_plane), tab(xz_plane), tab(yz_plane), B, nw)
